# Initial kernel scaffold; baseline (speedup 1.0000x reference)
import jax
import jax.numpy as jnp
from jax.experimental import pallas as pl


def kernel(nei, h, h_refer, att):
    raise NotImplementedError("write your pallas kernel here")


# speedup vs baseline: 1.3946x; 1.3946x over previous
import functools

import jax
import jax.numpy as jnp
from jax import lax
from jax.experimental import pallas as pl
from jax.experimental.pallas import tpu as pltpu
from jax.experimental.pallas import tpu_sc as plsc

_HID = 128
_NEI = 32
_G = 4
_L = 16


@functools.lru_cache(maxsize=None)
def _build(n_pad, n_nodes, ng):
  nw = 32
  rpw = n_pad // nw
  mesh = plsc.VectorSubcoreMesh(core_axis_name="c", subcore_axis_name="s",
                                num_cores=2, num_subcores=16)

  def body(nei_hbm, h_hbm, out_hbm, idx0, idx1, rows0, rows1, out_v,
           sem0, sem1):
    wid = lax.axis_index("s") * 2 + lax.axis_index("c")
    row_base = wid * rpw

    def start(g, idx_v, rows_v, sem):
      ibase = (row_base + g * _G) * _NEI
      pltpu.sync_copy(nei_hbm.at[pl.ds(ibase, _G * _NEI)], idx_v)
      return pltpu.async_copy(h_hbm.at[idx_v], rows_v, sem)

    def reduce_group(g, rows_v):
      def jstep(j, accs):
        new = []
        for d in range(_G):
          r = d * _NEI + j
          for c in range(_HID // _L):
            new.append(accs[d * (_HID // _L) + c]
                       + rows_v[r, pl.ds(c * _L, _L)])
        return tuple(new)

      init = tuple(jnp.zeros((_L,), jnp.float32)
                   for _ in range(_G * (_HID // _L)))
      accs = lax.fori_loop(0, _NEI, jstep, init)
      inv = jnp.float32(1.0 / _NEI)
      for d in range(_G):
        for c in range(_HID // _L):
          out_v[d, pl.ds(c * _L, _L)] = accs[d * (_HID // _L) + c] * inv
      pltpu.sync_copy(out_v, out_hbm.at[pl.ds(row_base + g * _G, _G)])

    start(0, idx0, rows0, sem0)
    start(1, idx1, rows1, sem1)

    def outer(i, _):
      g = i * 2
      for b, (idx_v, rows_v, sem) in enumerate(
          ((idx0, rows0, sem0), (idx1, rows1, sem1))):
        gb = g + b
        pltpu.make_async_copy(h_hbm.at[idx_v], rows_v, sem).wait()
        reduce_group(gb, rows_v)

        @pl.when(gb + 2 < ng)
        def _():
          start(gb + 2, idx_v, rows_v, sem)
      return 0

    lax.fori_loop(0, ng // 2, outer, 0)

  grid_kernel = pl.kernel(
      body,
      out_type=jax.ShapeDtypeStruct((n_pad, _HID), jnp.float32),
      mesh=mesh,
      scratch_types=[
          pltpu.VMEM((_G * _NEI,), jnp.int32),
          pltpu.VMEM((_G * _NEI,), jnp.int32),
          pltpu.VMEM((_G * _NEI, _HID), jnp.float32),
          pltpu.VMEM((_G * _NEI, _HID), jnp.float32),
          pltpu.VMEM((_G, _HID), jnp.float32),
          pltpu.SemaphoreType.DMA,
          pltpu.SemaphoreType.DMA,
      ],
  )
  return grid_kernel


@jax.jit
def kernel(nei, h, h_refer, att):
  n_dst = nei.shape[0]
  nw = 32
  per_step = nw * _G
  ng = -(-n_dst // per_step)
  if ng % 2:
    ng += 1
  n_pad = ng * per_step
  nei_flat = jnp.pad(nei.astype(jnp.int32),
                     ((0, n_pad - n_dst), (0, 0))).reshape(-1)
  out = _build(n_pad, h.shape[0], ng)(nei_flat, h)
  return out[:n_dst]


# speedup vs baseline: 2.7168x; 1.9480x over previous
import functools

import jax
import jax.numpy as jnp
from jax import lax
from jax.experimental import pallas as pl
from jax.experimental.pallas import tpu as pltpu
from jax.experimental.pallas import tpu_sc as plsc

_HID = 128
_NEI = 32
_G = 4
_L = 16
_NSUB = 16
_NG0 = 124
_NG1 = 34


@functools.lru_cache(maxsize=None)
def _build(n_pad, n_nodes):
  pair_rows = (_NG0 + _NG1) * _G
  assert n_pad == _NSUB * pair_rows
  mesh = plsc.VectorSubcoreMesh(core_axis_name="c", subcore_axis_name="s",
                                num_cores=2, num_subcores=16)

  def body(nei_hbm, h_hbm, out_hbm, idx0, idx1, rows0, rows1, out_v,
           sem0, sem1):
    c = lax.axis_index("c")
    s = lax.axis_index("s")
    row_base = s * pair_rows + jnp.where(c == 0, 0, _NG0 * _G)
    ng = jnp.where(c == 0, _NG0, _NG1)

    def start(g, idx_v, rows_v, sem):
      ibase = (row_base + g * _G) * _NEI
      pltpu.sync_copy(nei_hbm.at[pl.ds(ibase, _G * _NEI)], idx_v)
      return pltpu.async_copy(h_hbm.at[idx_v], rows_v, sem)

    def reduce_group(g, rows_v):
      def jstep(j, accs):
        new = []
        for d in range(_G):
          r = d * _NEI + j
          for k in range(_HID // _L):
            new.append(accs[d * (_HID // _L) + k]
                       + rows_v[r, pl.ds(k * _L, _L)])
        return tuple(new)

      init = tuple(jnp.zeros((_L,), jnp.float32)
                   for _ in range(_G * (_HID // _L)))
      accs = lax.fori_loop(0, _NEI, jstep, init)
      inv = jnp.float32(1.0 / _NEI)
      for d in range(_G):
        for k in range(_HID // _L):
          out_v[d, pl.ds(k * _L, _L)] = accs[d * (_HID // _L) + k] * inv
      pltpu.sync_copy(out_v, out_hbm.at[pl.ds(row_base + g * _G, _G)])

    start(0, idx0, rows0, sem0)
    start(1, idx1, rows1, sem1)

    def outer(i, _):
      g = i * 2
      for b, (idx_v, rows_v, sem) in enumerate(
          ((idx0, rows0, sem0), (idx1, rows1, sem1))):
        gb = g + b
        pltpu.make_async_copy(h_hbm.at[idx_v], rows_v, sem).wait()
        reduce_group(gb, rows_v)

        @pl.when(gb + 2 < ng)
        def _():
          start(gb + 2, idx_v, rows_v, sem)
      return 0

    lax.fori_loop(0, ng // 2, outer, 0)

  grid_kernel = pl.kernel(
      body,
      out_type=jax.ShapeDtypeStruct((n_pad, _HID), jnp.float32),
      mesh=mesh,
      scratch_types=[
          pltpu.VMEM((_G * _NEI,), jnp.int32),
          pltpu.VMEM((_G * _NEI,), jnp.int32),
          pltpu.VMEM((_G * _NEI, _HID), jnp.float32),
          pltpu.VMEM((_G * _NEI, _HID), jnp.float32),
          pltpu.VMEM((_G, _HID), jnp.float32),
          pltpu.SemaphoreType.DMA,
          pltpu.SemaphoreType.DMA,
      ],
  )
  return grid_kernel


@jax.jit
def kernel(nei, h, h_refer, att):
  n_dst = nei.shape[0]
  n_pad = _NSUB * (_NG0 + _NG1) * _G
  nei_flat = jnp.pad(nei.astype(jnp.int32),
                     ((0, n_pad - n_dst), (0, 0))).reshape(-1)
  out = _build(n_pad, h.shape[0])(nei_flat, h)
  return out[:n_dst]
